# baseline (speedup 1.0000x reference)
import functools
import math

import jax
import jax.numpy as jnp
from jax import lax
from jax.experimental import pallas as pl
from jax.experimental.pallas import tpu as pltpu
from jax.experimental.pallas import tpu_sc as plsc

TIME = 288
F = 256
B = 64
N = 883
NPAD = 896
NFULL = 55
NTAIL = N - 16
TWO_PI = 2.0 * math.pi


def _prologue_body(d_ref, w_ref, tw_ref, bcd_ref, bcw_ref,
                   didx_ref, widx_ref, sd_ref, sw_ref, tw2_ref):
    d = d_ref[...]
    w = w_ref[...]
    di = jnp.clip((d * float(TIME)).astype(jnp.int32), 0, TIME - 1)
    didx_ref[...] = di * F
    wi = jnp.clip(w.astype(jnp.int32), 0, 6)
    widx_ref[...] = wi * F
    ad = d * (TWO_PI / TIME)
    aw = w * (TWO_PI / 7.0)
    sd_ref[...] = jnp.sin(ad) + jnp.cos(ad)
    sw_ref[...] = jnp.sin(aw) + jnp.cos(aw)
    tw2_ref[...] = tw_ref[...] + bcd_ref[...] + bcw_ref[...]


def _sc_body(tday, tweek, wcd, wcw, didx, widx, sdh, swh, out_hbm,
             tday_v, tweek_v, wcd_v, wcw_v, didx_v, widx_v, sd_v, sw_v, obuf):
    cid = lax.axis_index("c")
    sid = lax.axis_index("s")
    wid = sid * 2 + cid
    pltpu.sync_copy(tday, tday_v)
    pltpu.sync_copy(tweek, tweek_v)
    pltpu.sync_copy(wcd, wcd_v)
    pltpu.sync_copy(wcw, wcw_v)

    for bi in range(2):
        b = wid * 2 + bi
        pltpu.sync_copy(didx.at[b], didx_v)
        pltpu.sync_copy(widx.at[b], widx_v)
        pltpu.sync_copy(sdh.at[b], sd_v)
        pltpu.sync_copy(swh.at[b], sw_v)

        def octet_body(o, carry_b):
            f0 = o * 8
            splats = []
            for f in range(8):
                fidx = jnp.full((16,), f0 + f, jnp.int32)
                wcd_s = plsc.load_gather(wcd_v, [fidx])
                wcw_s = plsc.load_gather(wcw_v, [fidx])
                splats.append((fidx, wcd_s, wcw_s))

            def do_chunk(off):
                dix = didx_v[pl.ds(off, 16)]
                wix = widx_v[pl.ds(off, 16)]
                sdc = sd_v[pl.ds(off, 16)]
                swc = sw_v[pl.ds(off, 16)]
                for f in range(8):
                    fidx, wcd_s, wcw_s = splats[f]
                    gd = plsc.load_gather(tday_v, [dix + fidx])
                    gw = plsc.load_gather(tweek_v, [wix + fidx])
                    val = gd + gw + sdc * wcd_s + swc * wcw_s
                    obuf[f, pl.ds(off, 16)] = val

            def chunk_body(c, carry_c):
                do_chunk(c * 16)
                return carry_c

            lax.fori_loop(0, NFULL, chunk_body, 0, unroll=False)
            do_chunk(NTAIL)
            pltpu.sync_copy(obuf, out_hbm.at[carry_b, pl.ds(f0, 8)])
            return carry_b

        lax.fori_loop(0, F // 8, octet_body, b, unroll=False)


def _build_sc_kernel():
    mesh = plsc.VectorSubcoreMesh(core_axis_name="c", subcore_axis_name="s")
    return functools.partial(
        pl.kernel,
        mesh=mesh,
        out_type=jax.ShapeDtypeStruct((B, F, N), jnp.float32),
        compiler_params=pltpu.CompilerParams(needs_layout_passes=False),
        scratch_types=[
            pltpu.VMEM((TIME * F,), jnp.float32),
            pltpu.VMEM((8 * F,), jnp.float32),
            pltpu.VMEM((F,), jnp.float32),
            pltpu.VMEM((F,), jnp.float32),
            pltpu.VMEM((NPAD,), jnp.int32),
            pltpu.VMEM((NPAD,), jnp.int32),
            pltpu.VMEM((NPAD,), jnp.float32),
            pltpu.VMEM((NPAD,), jnp.float32),
            pltpu.VMEM((8, N), jnp.float32),
        ],
    )(_sc_body)


_sc_kernel = _build_sc_kernel()


@jax.jit
def kernel(x, time_day, time_week, W_cd, b_cd, W_cw, b_cw):
    d = x[:, -1, :, 1]
    w = x[:, -1, :, 2]
    pad = ((0, 0), (0, NPAD - N))
    d = jnp.pad(d, pad)
    w = jnp.pad(w, pad)
    tw_pad = jnp.pad(time_week, ((0, 1), (0, 0)))

    didx, widx, sd, sw, tw2 = pl.pallas_call(
        _prologue_body,
        out_shape=(
            jax.ShapeDtypeStruct((B, NPAD), jnp.int32),
            jax.ShapeDtypeStruct((B, NPAD), jnp.int32),
            jax.ShapeDtypeStruct((B, NPAD), jnp.float32),
            jax.ShapeDtypeStruct((B, NPAD), jnp.float32),
            jax.ShapeDtypeStruct((8, F), jnp.float32),
        ),
    )(d, w, tw_pad, b_cd.reshape(1, F), b_cw.reshape(1, F))

    out = _sc_kernel(
        time_day.reshape(-1),
        tw2.reshape(-1),
        W_cd.reshape(-1),
        W_cw.reshape(-1),
        didx,
        widx,
        sd,
        sw,
    )
    return out[..., None]


# speedup vs baseline: 1.6134x; 1.6134x over previous
import functools
import math

import jax
import jax.numpy as jnp
from jax import lax
from jax.experimental import pallas as pl
from jax.experimental.pallas import tpu as pltpu
from jax.experimental.pallas import tpu_sc as plsc

TIME = 288
F = 256
B = 64
N = 883
NPAD = 896
NFULL = 55
NTAIL = N - 16
TWO_PI = 2.0 * math.pi


def _prologue_body(d_ref, w_ref, tw_ref, bcd_ref, bcw_ref,
                   didx_ref, widx_ref, sd_ref, sw_ref, tw2_ref):
    d = d_ref[...]
    w = w_ref[...]
    di = jnp.clip((d * float(TIME)).astype(jnp.int32), 0, TIME - 1)
    didx_ref[...] = di * F
    wi = jnp.clip(w.astype(jnp.int32), 0, 6)
    widx_ref[...] = wi * F
    ad = d * (TWO_PI / TIME)
    aw = w * (TWO_PI / 7.0)
    sd_ref[...] = jnp.sin(ad) + jnp.cos(ad)
    sw_ref[...] = jnp.sin(aw) + jnp.cos(aw)
    tw2_ref[...] = tw_ref[...] + bcd_ref[...] + bcw_ref[...]


def _sc_body(tday, tweek, wcd, wcw, didx, widx, sdh, swh, out_hbm,
             tday_v, tweek_v, wcd_v, wcw_v, didx_v, widx_v, sd_v, sw_v, obuf,
             osem):
    cid = lax.axis_index("c")
    sid = lax.axis_index("s")
    wid = sid * 2 + cid
    pltpu.sync_copy(tday, tday_v)
    pltpu.sync_copy(tweek, tweek_v)
    pltpu.sync_copy(wcd, wcd_v)
    pltpu.sync_copy(wcw, wcw_v)

    for bi in range(2):
        b = wid * 2 + bi
        pltpu.sync_copy(didx.at[b], didx_v)
        pltpu.sync_copy(widx.at[b], widx_v)
        pltpu.sync_copy(sdh.at[b], sd_v)
        pltpu.sync_copy(swh.at[b], sw_v)

        def octet_body(o, carry_b):
            f0 = o * 8
            slot = lax.rem(o, 2)
            splats = []
            for f in range(8):
                fidx = jnp.full((16,), f0 + f, jnp.int32)
                wcd_s = plsc.load_gather(wcd_v, [fidx])
                wcw_s = plsc.load_gather(wcw_v, [fidx])
                splats.append((fidx, wcd_s, wcw_s))

            @pl.when(o >= 2)
            def _wait_prev():
                pltpu.make_async_copy(
                    obuf.at[slot], out_hbm.at[carry_b, pl.ds(f0, 8)], osem
                ).wait()

            def do_chunk(off):
                dix = didx_v[pl.ds(off, 16)]
                wix = widx_v[pl.ds(off, 16)]
                sdc = sd_v[pl.ds(off, 16)]
                swc = sw_v[pl.ds(off, 16)]
                idxs = [(dix + s[0], wix + s[0]) for s in splats]
                gds = [plsc.load_gather(tday_v, [i[0]]) for i in idxs]
                gws = [plsc.load_gather(tweek_v, [i[1]]) for i in idxs]
                vals = []
                for f in range(8):
                    _, wcd_s, wcw_s = splats[f]
                    vals.append((gds[f] + gws[f]) + (sdc * wcd_s + swc * wcw_s))
                for f in range(8):
                    obuf[slot, f, pl.ds(off, 16)] = vals[f]

            @plsc.parallel_loop(0, NFULL, 1, unroll=2)
            def _chunks(c):
                do_chunk(c * 16)

            do_chunk(NTAIL)
            pltpu.async_copy(
                obuf.at[slot], out_hbm.at[carry_b, pl.ds(f0, 8)], osem
            )
            return carry_b

        lax.fori_loop(0, F // 8, octet_body, b, unroll=False)
        for s in range(2):
            pltpu.make_async_copy(
                obuf.at[s], out_hbm.at[b, pl.ds(0, 8)], osem
            ).wait()


def _build_sc_kernel():
    mesh = plsc.VectorSubcoreMesh(core_axis_name="c", subcore_axis_name="s")
    return functools.partial(
        pl.kernel,
        mesh=mesh,
        out_type=jax.ShapeDtypeStruct((B, F, N), jnp.float32),
        compiler_params=pltpu.CompilerParams(needs_layout_passes=False),
        scratch_types=[
            pltpu.VMEM((TIME * F,), jnp.float32),
            pltpu.VMEM((8 * F,), jnp.float32),
            pltpu.VMEM((F,), jnp.float32),
            pltpu.VMEM((F,), jnp.float32),
            pltpu.VMEM((NPAD,), jnp.int32),
            pltpu.VMEM((NPAD,), jnp.int32),
            pltpu.VMEM((NPAD,), jnp.float32),
            pltpu.VMEM((NPAD,), jnp.float32),
            pltpu.VMEM((2, 8, N), jnp.float32),
            pltpu.SemaphoreType.DMA,
        ],
    )(_sc_body)


_sc_kernel = _build_sc_kernel()


@jax.jit
def kernel(x, time_day, time_week, W_cd, b_cd, W_cw, b_cw):
    d = x[:, -1, :, 1]
    w = x[:, -1, :, 2]
    pad = ((0, 0), (0, NPAD - N))
    d = jnp.pad(d, pad)
    w = jnp.pad(w, pad)
    tw_pad = jnp.pad(time_week, ((0, 1), (0, 0)))

    didx, widx, sd, sw, tw2 = pl.pallas_call(
        _prologue_body,
        out_shape=(
            jax.ShapeDtypeStruct((B, NPAD), jnp.int32),
            jax.ShapeDtypeStruct((B, NPAD), jnp.int32),
            jax.ShapeDtypeStruct((B, NPAD), jnp.float32),
            jax.ShapeDtypeStruct((B, NPAD), jnp.float32),
            jax.ShapeDtypeStruct((8, F), jnp.float32),
        ),
    )(d, w, tw_pad, b_cd.reshape(1, F), b_cw.reshape(1, F))

    out = _sc_kernel(
        time_day.reshape(-1),
        tw2.reshape(-1),
        W_cd.reshape(-1),
        W_cw.reshape(-1),
        didx,
        widx,
        sd,
        sw,
    )
    return out[..., None]


# speedup vs baseline: 1.6976x; 1.0522x over previous
import functools
import math

import jax
import jax.numpy as jnp
from jax import lax
from jax.experimental import pallas as pl
from jax.experimental.pallas import tpu as pltpu
from jax.experimental.pallas import tpu_sc as plsc

TIME = 288
F = 256
B = 64
N = 883
NPAD = 896
NFULL = 55
NTAIL = N - 16
TWO_PI = 2.0 * math.pi


def _prologue_body(d_ref, w_ref, tw_ref, bcd_ref, bcw_ref,
                   didx_ref, widx_ref, sd_ref, sw_ref, tw2_ref):
    d = d_ref[...]
    w = w_ref[...]
    di = jnp.clip((d * float(TIME)).astype(jnp.int32), 0, TIME - 1)
    didx_ref[...] = di * F
    wi = jnp.clip(w.astype(jnp.int32), 0, 6)
    widx_ref[...] = wi * F
    ad = d * (TWO_PI / TIME)
    aw = w * (TWO_PI / 7.0)
    sd_ref[...] = jnp.sin(ad) + jnp.cos(ad)
    sw_ref[...] = jnp.sin(aw) + jnp.cos(aw)
    tw2_ref[...] = tw_ref[...] + bcd_ref[...] + bcw_ref[...]


def _sc_body(tday, tweek, wcd, wcw, didx, widx, sdh, swh, out_hbm,
             tday_v, tweek_v, wcd_v, wcw_v, didx_v, widx_v, sd_v, sw_v, obuf,
             osem):
    cid = lax.axis_index("c")
    sid = lax.axis_index("s")
    wid = sid * 2 + cid
    pltpu.sync_copy(tday, tday_v)
    pltpu.sync_copy(tweek, tweek_v)
    pltpu.sync_copy(wcd, wcd_v)
    pltpu.sync_copy(wcw, wcw_v)

    for bi in range(2):
        b = wid * 2 + bi
        pltpu.sync_copy(didx.at[b], didx_v)
        pltpu.sync_copy(widx.at[b], widx_v)
        pltpu.sync_copy(sdh.at[b], sd_v)
        pltpu.sync_copy(swh.at[b], sw_v)

        def octet_body(o, carry_b):
            f0 = o * 8
            slot = lax.rem(o, 2)
            f0_splat = jnp.full((16,), f0, jnp.int32)
            splats = []
            for f in range(8):
                fidx = f0_splat + f
                wcd_s = plsc.load_gather(wcd_v, [fidx])
                wcw_s = plsc.load_gather(wcw_v, [fidx])
                splats.append((wcd_s, wcw_s))

            @pl.when(o >= 2)
            def _wait_prev():
                pltpu.make_async_copy(
                    obuf.at[slot], out_hbm.at[carry_b, pl.ds(f0, 8)], osem
                ).wait()

            def do_chunk(off):
                dix = didx_v[pl.ds(off, 16)]
                wix = widx_v[pl.ds(off, 16)]
                sdc = sd_v[pl.ds(off, 16)]
                swc = sw_v[pl.ds(off, 16)]
                base_d = dix + f0_splat
                base_w = wix + f0_splat
                for q in range(2):
                    gds = [plsc.load_gather(tday_v, [base_d + f])
                           for f in range(q * 4, q * 4 + 4)]
                    gws = [plsc.load_gather(tweek_v, [base_w + f])
                           for f in range(q * 4, q * 4 + 4)]
                    for j in range(4):
                        f = q * 4 + j
                        wcd_s, wcw_s = splats[f]
                        val = (gds[j] + gws[j]) + (sdc * wcd_s + swc * wcw_s)
                        obuf[slot, f, pl.ds(off, 16)] = val

            @plsc.parallel_loop(0, NFULL, 1, unroll=1)
            def _chunks(c):
                do_chunk(c * 16)

            do_chunk(NTAIL)
            pltpu.async_copy(
                obuf.at[slot], out_hbm.at[carry_b, pl.ds(f0, 8)], osem
            )
            return carry_b

        lax.fori_loop(0, F // 8, octet_body, b, unroll=False)
        for s in range(2):
            pltpu.make_async_copy(
                obuf.at[s], out_hbm.at[b, pl.ds(0, 8)], osem
            ).wait()


def _build_sc_kernel():
    mesh = plsc.VectorSubcoreMesh(core_axis_name="c", subcore_axis_name="s")
    return functools.partial(
        pl.kernel,
        mesh=mesh,
        out_type=jax.ShapeDtypeStruct((B, F, N), jnp.float32),
        compiler_params=pltpu.CompilerParams(needs_layout_passes=False),
        scratch_types=[
            pltpu.VMEM((TIME * F,), jnp.float32),
            pltpu.VMEM((8 * F,), jnp.float32),
            pltpu.VMEM((F,), jnp.float32),
            pltpu.VMEM((F,), jnp.float32),
            pltpu.VMEM((NPAD,), jnp.int32),
            pltpu.VMEM((NPAD,), jnp.int32),
            pltpu.VMEM((NPAD,), jnp.float32),
            pltpu.VMEM((NPAD,), jnp.float32),
            pltpu.VMEM((2, 8, N), jnp.float32),
            pltpu.SemaphoreType.DMA,
        ],
    )(_sc_body)


_sc_kernel = _build_sc_kernel()


@jax.jit
def kernel(x, time_day, time_week, W_cd, b_cd, W_cw, b_cw):
    d = x[:, -1, :, 1]
    w = x[:, -1, :, 2]
    pad = ((0, 0), (0, NPAD - N))
    d = jnp.pad(d, pad)
    w = jnp.pad(w, pad)
    tw_pad = jnp.pad(time_week, ((0, 1), (0, 0)))

    didx, widx, sd, sw, tw2 = pl.pallas_call(
        _prologue_body,
        out_shape=(
            jax.ShapeDtypeStruct((B, NPAD), jnp.int32),
            jax.ShapeDtypeStruct((B, NPAD), jnp.int32),
            jax.ShapeDtypeStruct((B, NPAD), jnp.float32),
            jax.ShapeDtypeStruct((B, NPAD), jnp.float32),
            jax.ShapeDtypeStruct((8, F), jnp.float32),
        ),
    )(d, w, tw_pad, b_cd.reshape(1, F), b_cw.reshape(1, F))

    out = _sc_kernel(
        time_day.reshape(-1),
        tw2.reshape(-1),
        W_cd.reshape(-1),
        W_cw.reshape(-1),
        didx,
        widx,
        sd,
        sw,
    )
    return out[..., None]


# speedup vs baseline: 3.8414x; 2.2628x over previous
import functools
import math

import jax
import jax.numpy as jnp
from jax import lax
from jax.experimental import pallas as pl
from jax.experimental.pallas import tpu as pltpu
from jax.experimental.pallas import tpu_sc as plsc

TIME = 288
F = 256
B = 64
N = 883
NPAD = 896
NFULL = 55
NTAIL = N - 16
TWO_PI = 2.0 * math.pi


def _prologue_body(d_ref, w_ref, td_ref, tw_ref, bcd_ref, bcw_ref,
                   didx_ref, widx_ref, sd_ref, sw_ref, tdt_ref, tw2_ref):
    d = d_ref[...]
    w = w_ref[...]
    di = jnp.clip((d * float(TIME)).astype(jnp.int32), 0, TIME - 1)
    didx_ref[...] = di
    wi = jnp.clip(w.astype(jnp.int32), 0, 6)
    widx_ref[...] = wi * F
    ad = d * (TWO_PI / TIME)
    aw = w * (TWO_PI / 7.0)
    sd_ref[...] = jnp.sin(ad) + jnp.cos(ad)
    sw_ref[...] = jnp.sin(aw) + jnp.cos(aw)
    tdt_ref[...] = td_ref[...].T
    tw2_ref[...] = tw_ref[...] + bcd_ref[...] + bcw_ref[...]


def _sc_body(tday, tweek, wcd, wcw, didx, widx, sdh, swh, out_hbm,
             tday_v, tweek_v, wcd_v, wcw_v, didx_v, widx_v, sd_v, sw_v, obuf,
             osem):
    cid = lax.axis_index("c")
    sid = lax.axis_index("s")
    wid = sid * 2 + cid
    pltpu.sync_copy(tday, tday_v)
    pltpu.sync_copy(tweek, tweek_v)
    pltpu.sync_copy(wcd, wcd_v)
    pltpu.sync_copy(wcw, wcw_v)

    for bi in range(2):
        b = wid * 2 + bi
        pltpu.sync_copy(didx.at[b], didx_v)
        pltpu.sync_copy(widx.at[b], widx_v)
        pltpu.sync_copy(sdh.at[b], sd_v)
        pltpu.sync_copy(swh.at[b], sw_v)

        def octet_body(o, carry_b):
            f0 = o * 8
            slot = lax.rem(o, 2)
            f0_splat = jnp.full((16,), f0, jnp.int32)
            f0_splat288 = jnp.full((16,), f0 * TIME, jnp.int32)
            splats = []
            for f in range(8):
                fidx = f0_splat + f
                wcd_s = plsc.load_gather(wcd_v, [fidx])
                wcw_s = plsc.load_gather(wcw_v, [fidx])
                splats.append((wcd_s, wcw_s))

            @pl.when(o >= 2)
            def _wait_prev():
                pltpu.make_async_copy(
                    obuf.at[slot], out_hbm.at[carry_b, pl.ds(f0, 8)], osem
                ).wait()

            def do_chunk(off):
                dix = didx_v[pl.ds(off, 16)]
                wix = widx_v[pl.ds(off, 16)]
                sdc = sd_v[pl.ds(off, 16)]
                swc = sw_v[pl.ds(off, 16)]
                base_d = dix + f0_splat288
                base_w = wix + f0_splat
                for q in range(2):
                    gds = [plsc.load_gather(tday_v, [base_d + f * TIME])
                           for f in range(q * 4, q * 4 + 4)]
                    gws = [plsc.load_gather(tweek_v, [base_w + f])
                           for f in range(q * 4, q * 4 + 4)]
                    for j in range(4):
                        f = q * 4 + j
                        wcd_s, wcw_s = splats[f]
                        val = (gds[j] + gws[j]) + (sdc * wcd_s + swc * wcw_s)
                        obuf[slot, f, pl.ds(off, 16)] = val

            @plsc.parallel_loop(0, NFULL, 1, unroll=1)
            def _chunks(c):
                do_chunk(c * 16)

            do_chunk(NTAIL)
            pltpu.async_copy(
                obuf.at[slot], out_hbm.at[carry_b, pl.ds(f0, 8)], osem
            )
            return carry_b

        lax.fori_loop(0, F // 8, octet_body, b, unroll=False)
        for s in range(2):
            pltpu.make_async_copy(
                obuf.at[s], out_hbm.at[b, pl.ds(0, 8)], osem
            ).wait()


def _build_sc_kernel():
    mesh = plsc.VectorSubcoreMesh(core_axis_name="c", subcore_axis_name="s")
    return functools.partial(
        pl.kernel,
        mesh=mesh,
        out_type=jax.ShapeDtypeStruct((B, F, N), jnp.float32),
        compiler_params=pltpu.CompilerParams(needs_layout_passes=False),
        scratch_types=[
            pltpu.VMEM((TIME * F,), jnp.float32),
            pltpu.VMEM((8 * F,), jnp.float32),
            pltpu.VMEM((F,), jnp.float32),
            pltpu.VMEM((F,), jnp.float32),
            pltpu.VMEM((NPAD,), jnp.int32),
            pltpu.VMEM((NPAD,), jnp.int32),
            pltpu.VMEM((NPAD,), jnp.float32),
            pltpu.VMEM((NPAD,), jnp.float32),
            pltpu.VMEM((2, 8, N), jnp.float32),
            pltpu.SemaphoreType.DMA,
        ],
    )(_sc_body)


_sc_kernel = _build_sc_kernel()


@jax.jit
def kernel(x, time_day, time_week, W_cd, b_cd, W_cw, b_cw):
    d = x[:, -1, :, 1]
    w = x[:, -1, :, 2]
    pad = ((0, 0), (0, NPAD - N))
    d = jnp.pad(d, pad)
    w = jnp.pad(w, pad)
    tw_pad = jnp.pad(time_week, ((0, 1), (0, 0)))

    didx, widx, sd, sw, tdt, tw2 = pl.pallas_call(
        _prologue_body,
        out_shape=(
            jax.ShapeDtypeStruct((B, NPAD), jnp.int32),
            jax.ShapeDtypeStruct((B, NPAD), jnp.int32),
            jax.ShapeDtypeStruct((B, NPAD), jnp.float32),
            jax.ShapeDtypeStruct((B, NPAD), jnp.float32),
            jax.ShapeDtypeStruct((F, TIME), jnp.float32),
            jax.ShapeDtypeStruct((8, F), jnp.float32),
        ),
    )(d, w, time_day, tw_pad, b_cd.reshape(1, F), b_cw.reshape(1, F))

    out = _sc_kernel(
        tdt.reshape(-1),
        tw2.reshape(-1),
        W_cd.reshape(-1),
        W_cw.reshape(-1),
        didx,
        widx,
        sd,
        sw,
    )
    return out[..., None]


# speedup vs baseline: 3.8916x; 1.0131x over previous
import functools
import math

import jax
import jax.numpy as jnp
from jax import lax
from jax.experimental import pallas as pl
from jax.experimental.pallas import tpu as pltpu
from jax.experimental.pallas import tpu_sc as plsc

TIME = 288
F = 256
B = 64
N = 883
NPAD = 896
NFULL = 55
NTAIL = N - 16
TWO_PI = 2.0 * math.pi


def _prologue_body(d_ref, w_ref, td_ref,
                   didx_ref, widx_ref, sd_ref, sw_ref, tdt_ref):
    d = d_ref[...]
    w = w_ref[...]
    di = jnp.clip((d * float(TIME)).astype(jnp.int32), 0, TIME - 1)
    didx_ref[...] = di
    wi = jnp.clip(w.astype(jnp.int32), 0, 6)
    widx_ref[...] = wi * (F * 16)
    ad = d * (TWO_PI / TIME)
    aw = w * (TWO_PI / 7.0)
    sd_ref[...] = jnp.sin(ad) + jnp.cos(ad)
    sw_ref[...] = jnp.sin(aw) + jnp.cos(aw)
    tdt_ref[...] = td_ref[...].T


def _sc_body(tday, tweek, wcd, wcw, didx, widx, sdh, swh, out_hbm,
             tday_v, tweek_v, wcd_v, wcw_v, didx_v, widx_v, sd_v, sw_v, obuf,
             osem):
    cid = lax.axis_index("c")
    sid = lax.axis_index("s")
    wid = sid * 2 + cid
    pltpu.sync_copy(tday, tday_v)
    pltpu.sync_copy(tweek, tweek_v)
    pltpu.sync_copy(wcd, wcd_v)
    pltpu.sync_copy(wcw, wcw_v)

    for bi in range(2):
        b = wid * 2 + bi
        pltpu.sync_copy(didx.at[b], didx_v)
        pltpu.sync_copy(widx.at[b], widx_v)
        pltpu.sync_copy(sdh.at[b], sd_v)
        pltpu.sync_copy(swh.at[b], sw_v)

        def octet_body(o, carry_b):
            f0 = o * 8
            slot = lax.rem(o, 2)
            f0_splat288 = jnp.full((16,), f0 * TIME, jnp.int32)
            base_s = lax.iota(jnp.int32, 16) + f0 * 16
            splats = []
            for f in range(8):
                fidx = base_s + f * 16
                wcd_s = plsc.load_gather(wcd_v, [fidx])
                wcw_s = plsc.load_gather(wcw_v, [fidx])
                splats.append((wcd_s, wcw_s))

            @pl.when(o >= 2)
            def _wait_prev():
                pltpu.make_async_copy(
                    obuf.at[slot], out_hbm.at[carry_b, pl.ds(f0, 8)], osem
                ).wait()

            def do_chunk(off):
                dix = didx_v[pl.ds(off, 16)]
                wix = widx_v[pl.ds(off, 16)]
                sdc = sd_v[pl.ds(off, 16)]
                swc = sw_v[pl.ds(off, 16)]
                base_d = dix + f0_splat288
                base_w = wix + base_s
                for q in range(2):
                    gds = [plsc.load_gather(tday_v, [base_d + f * TIME])
                           for f in range(q * 4, q * 4 + 4)]
                    gws = [plsc.load_gather(tweek_v, [base_w + f * 16])
                           for f in range(q * 4, q * 4 + 4)]
                    for j in range(4):
                        f = q * 4 + j
                        wcd_s, wcw_s = splats[f]
                        val = (gds[j] + gws[j]) + (sdc * wcd_s + swc * wcw_s)
                        obuf[slot, f, pl.ds(off, 16)] = val

            @plsc.parallel_loop(0, NFULL, 1, unroll=1)
            def _chunks(c):
                do_chunk(c * 16)

            do_chunk(NTAIL)
            pltpu.async_copy(
                obuf.at[slot], out_hbm.at[carry_b, pl.ds(f0, 8)], osem
            )
            return carry_b

        lax.fori_loop(0, F // 8, octet_body, b, unroll=False)
        for s in range(2):
            pltpu.make_async_copy(
                obuf.at[s], out_hbm.at[b, pl.ds(0, 8)], osem
            ).wait()


def _build_sc_kernel():
    mesh = plsc.VectorSubcoreMesh(core_axis_name="c", subcore_axis_name="s")
    return functools.partial(
        pl.kernel,
        mesh=mesh,
        out_type=jax.ShapeDtypeStruct((B, F, N), jnp.float32),
        compiler_params=pltpu.CompilerParams(needs_layout_passes=False),
        scratch_types=[
            pltpu.VMEM((TIME * F,), jnp.float32),
            pltpu.VMEM((7 * F * 16,), jnp.float32),
            pltpu.VMEM((F * 16,), jnp.float32),
            pltpu.VMEM((F * 16,), jnp.float32),
            pltpu.VMEM((NPAD,), jnp.int32),
            pltpu.VMEM((NPAD,), jnp.int32),
            pltpu.VMEM((NPAD,), jnp.float32),
            pltpu.VMEM((NPAD,), jnp.float32),
            pltpu.VMEM((2, 8, N), jnp.float32),
            pltpu.SemaphoreType.DMA,
        ],
    )(_sc_body)


_sc_kernel = _build_sc_kernel()


@jax.jit
def kernel(x, time_day, time_week, W_cd, b_cd, W_cw, b_cw):
    d = x[:, -1, :, 1]
    w = x[:, -1, :, 2]
    pad = ((0, 0), (0, NPAD - N))
    d = jnp.pad(d, pad)
    w = jnp.pad(w, pad)
    tw2 = time_week + b_cd.reshape(1, F) + b_cw.reshape(1, F)
    twr = jnp.repeat(tw2.reshape(-1), 16)
    wcdr = jnp.repeat(W_cd.reshape(-1), 16)
    wcwr = jnp.repeat(W_cw.reshape(-1), 16)

    didx, widx, sd, sw, tdt = pl.pallas_call(
        _prologue_body,
        out_shape=(
            jax.ShapeDtypeStruct((B, NPAD), jnp.int32),
            jax.ShapeDtypeStruct((B, NPAD), jnp.int32),
            jax.ShapeDtypeStruct((B, NPAD), jnp.float32),
            jax.ShapeDtypeStruct((B, NPAD), jnp.float32),
            jax.ShapeDtypeStruct((F, TIME), jnp.float32),
        ),
    )(d, w, time_day)

    out = _sc_kernel(
        tdt.reshape(-1),
        twr,
        wcdr,
        wcwr,
        didx,
        widx,
        sd,
        sw,
    )
    return out[..., None]


# speedup vs baseline: 3.9065x; 1.0038x over previous
import functools
import math

import jax
import jax.numpy as jnp
from jax import lax
from jax.experimental import pallas as pl
from jax.experimental.pallas import tpu as pltpu
from jax.experimental.pallas import tpu_sc as plsc

TIME = 288
F = 256
B = 64
N = 883
NPAD = 896
NFULL = 55
NTAIL = N - 16
TWO_PI = 2.0 * math.pi


def _prologue_body(d_ref, w_ref, td_ref,
                   didx_ref, widx_ref, sd_ref, sw_ref, tdt_ref):
    d = d_ref[...]
    w = w_ref[...]
    di = jnp.clip((d * float(TIME)).astype(jnp.int32), 0, TIME - 1)
    didx_ref[...] = di
    wi = jnp.clip(w.astype(jnp.int32), 0, 6)
    widx_ref[...] = wi * (F * 16)
    ad = d * (TWO_PI / TIME)
    aw = w * (TWO_PI / 7.0)
    sd_ref[...] = jnp.sin(ad) + jnp.cos(ad)
    sw_ref[...] = jnp.sin(aw) + jnp.cos(aw)
    tdt_ref[...] = td_ref[...].T


def _sc_body(tday, tweek, wcd, wcw, didx, widx, sdh, swh, out_hbm,
             tday_v, tweek_v, wcd_v, wcw_v, didx_v, widx_v, sd_v, sw_v, obuf,
             osem):
    cid = lax.axis_index("c")
    sid = lax.axis_index("s")
    wid = sid * 2 + cid
    pltpu.sync_copy(tday, tday_v)
    pltpu.sync_copy(tweek, tweek_v)
    pltpu.sync_copy(wcd, wcd_v)
    pltpu.sync_copy(wcw, wcw_v)

    for bi in range(2):
        b = wid * 2 + bi
        nb = b * NPAD
        pltpu.sync_copy(didx.at[pl.ds(nb, NPAD)], didx_v)
        pltpu.sync_copy(widx.at[pl.ds(nb, NPAD)], widx_v)
        pltpu.sync_copy(sdh.at[pl.ds(nb, NPAD)], sd_v)
        pltpu.sync_copy(swh.at[pl.ds(nb, NPAD)], sw_v)

        def octet_body(o, carry_b):
            f0 = o * 8
            slot = lax.rem(o, 2)
            f0_splat288 = jnp.full((16,), f0 * TIME, jnp.int32)
            base_s = lax.iota(jnp.int32, 16) + f0 * 16
            splats = []
            for f in range(8):
                fidx = base_s + f * 16
                wcd_s = plsc.load_gather(wcd_v, [fidx])
                wcw_s = plsc.load_gather(wcw_v, [fidx])
                splats.append((wcd_s, wcw_s))

            @pl.when(o >= 2)
            def _wait_prev():
                pltpu.make_async_copy(
                    obuf.at[slot], out_hbm.at[carry_b, pl.ds(f0, 8)], osem
                ).wait()

            def do_chunk(off):
                dix = didx_v[pl.ds(off, 16)]
                wix = widx_v[pl.ds(off, 16)]
                sdc = sd_v[pl.ds(off, 16)]
                swc = sw_v[pl.ds(off, 16)]
                base_d = dix + f0_splat288
                base_w = wix + base_s
                for q in range(2):
                    gds = [plsc.load_gather(tday_v, [base_d + f * TIME])
                           for f in range(q * 4, q * 4 + 4)]
                    gws = [plsc.load_gather(tweek_v, [base_w + f * 16])
                           for f in range(q * 4, q * 4 + 4)]
                    for j in range(4):
                        f = q * 4 + j
                        wcd_s, wcw_s = splats[f]
                        val = (gds[j] + gws[j]) + (sdc * wcd_s + swc * wcw_s)
                        obuf[slot, f, pl.ds(off, 16)] = val

            @plsc.parallel_loop(0, NFULL, 1, unroll=1)
            def _chunks(c):
                do_chunk(c * 16)

            do_chunk(NTAIL)
            pltpu.async_copy(
                obuf.at[slot], out_hbm.at[carry_b, pl.ds(f0, 8)], osem
            )
            return carry_b

        lax.fori_loop(0, F // 8, octet_body, b, unroll=False)
        for s in range(2):
            pltpu.make_async_copy(
                obuf.at[s], out_hbm.at[b, pl.ds(0, 8)], osem
            ).wait()


def _build_sc_kernel():
    mesh = plsc.VectorSubcoreMesh(core_axis_name="c", subcore_axis_name="s")
    return functools.partial(
        pl.kernel,
        mesh=mesh,
        out_type=jax.ShapeDtypeStruct((B, F, N), jnp.float32),
        compiler_params=pltpu.CompilerParams(needs_layout_passes=False),
        scratch_types=[
            pltpu.VMEM((TIME * F,), jnp.float32),
            pltpu.VMEM((7 * F * 16,), jnp.float32),
            pltpu.VMEM((F * 16,), jnp.float32),
            pltpu.VMEM((F * 16,), jnp.float32),
            pltpu.VMEM((NPAD,), jnp.int32),
            pltpu.VMEM((NPAD,), jnp.int32),
            pltpu.VMEM((NPAD,), jnp.float32),
            pltpu.VMEM((NPAD,), jnp.float32),
            pltpu.VMEM((2, 8, N), jnp.float32),
            pltpu.SemaphoreType.DMA,
        ],
    )(_sc_body)


_sc_kernel = _build_sc_kernel()


@jax.jit
def kernel(x, time_day, time_week, W_cd, b_cd, W_cw, b_cw):
    d = x[:, -1, :, 1]
    w = x[:, -1, :, 2]
    pad = ((0, 0), (0, NPAD - N))
    d = jnp.pad(d, pad)
    w = jnp.pad(w, pad)
    tw2 = time_week + b_cd.reshape(1, F) + b_cw.reshape(1, F)
    twr = jnp.repeat(tw2.reshape(-1), 16)
    wcdr = jnp.repeat(W_cd.reshape(-1), 16)
    wcwr = jnp.repeat(W_cw.reshape(-1), 16)

    didx, widx, sd, sw, tdt = pl.pallas_call(
        _prologue_body,
        out_shape=(
            jax.ShapeDtypeStruct((B * NPAD,), jnp.int32),
            jax.ShapeDtypeStruct((B * NPAD,), jnp.int32),
            jax.ShapeDtypeStruct((B * NPAD,), jnp.float32),
            jax.ShapeDtypeStruct((B * NPAD,), jnp.float32),
            jax.ShapeDtypeStruct((F, TIME), jnp.float32),
        ),
    )(d.reshape(-1), w.reshape(-1), time_day)

    out = _sc_kernel(
        tdt.reshape(-1),
        twr,
        wcdr,
        wcwr,
        didx,
        widx,
        sd,
        sw,
    )
    return out[..., None]
